# Initial kernel scaffold; baseline (speedup 1.0000x reference)
import jax
import jax.numpy as jnp
from jax.experimental import pallas as pl


def kernel(x, W0, W1, W2, W3, W4):
    raise NotImplementedError("write your pallas kernel here")


# speedup vs baseline: 3.4455x; 3.4455x over previous
import functools

import jax
import jax.numpy as jnp
from jax import lax
from jax.experimental import pallas as pl
from jax.experimental.pallas import tpu as pltpu
from jax.experimental.pallas import tpu_sc as plsc

EMB = 128
R0 = 119
RC = 288
RF = R0 * RC
MAX_NORM = 10.0

NC = 2
NS = 16
NW = NC * NS
GROUP = 128
GROUPS_PER_W = 25
PER_W = GROUP * GROUPS_PER_W
N_PAD = NW * PER_W

W0_BLK = 8


def _renorm(w):
    norm = jnp.sqrt(jnp.sum(w * w, axis=-1, keepdims=True))
    scale = jnp.where(norm > MAX_NORM, MAX_NORM / (norm + 1e-7), 1.0)
    return w * scale


def _build_body(w0_ref, w1_ref, w2_ref, w3_ref, w4_ref, out_ref, combo_ref):
    i = pl.program_id(0)

    @pl.when(i == 0)
    def _():
        r1 = _renorm(w1_ref[...])
        r2 = _renorm(w2_ref[...])
        r3 = _renorm(w3_ref[...])
        r4 = _renorm(w4_ref[...])
        r34 = jnp.concatenate([r3[0:1] + r4, r3[1:2] + r4], axis=0)
        r234 = jnp.concatenate([r2[k:k + 1] + r34 for k in range(6)], axis=0)
        combo_ref[...] = jnp.concatenate(
            [r1[a:a + 1] + r234 for a in range(12)], axis=0)

    r0 = _renorm(w0_ref[...])
    out_ref[...] = r0[:, None, :] + combo_ref[...][None, :, :]


def _build_table(W0, W1, W2, W3, W4):
    grid = (pl.cdiv(R0, W0_BLK),)
    out = pl.pallas_call(
        _build_body,
        grid=grid,
        in_specs=[
            pl.BlockSpec((W0_BLK, EMB), lambda i: (i, 0)),
            pl.BlockSpec((12, EMB), lambda i: (0, 0)),
            pl.BlockSpec((6, EMB), lambda i: (0, 0)),
            pl.BlockSpec((2, EMB), lambda i: (0, 0)),
            pl.BlockSpec((2, EMB), lambda i: (0, 0)),
        ],
        out_specs=pl.BlockSpec((W0_BLK, RC, EMB), lambda i: (i, 0, 0)),
        out_shape=jax.ShapeDtypeStruct((R0, RC, EMB), jnp.float32),
        scratch_shapes=[pltpu.VMEM((RC, EMB), jnp.float32)],
    )(W0, W1, W2, W3, W4)
    return out.reshape(RF, EMB)


def _sc_lookup_body(x0h, x1h, x2h, x3h, x4h, th, outh,
                    xb0, xb1, xb2, xb3, xb4, idx1, rows, sem):
    wid = lax.axis_index("s") * NC + lax.axis_index("c")
    base = wid * PER_W
    pltpu.sync_copy(x0h.at[pl.ds(base, PER_W)], xb0)
    pltpu.sync_copy(x1h.at[pl.ds(base, PER_W)], xb1)
    pltpu.sync_copy(x2h.at[pl.ds(base, PER_W)], xb2)
    pltpu.sync_copy(x3h.at[pl.ds(base, PER_W)], xb3)
    pltpu.sync_copy(x4h.at[pl.ds(base, PER_W)], xb4)

    def g_body(j, carry):
        for jj in range(GROUP // 16):
            p = j * GROUP + jj * 16
            v = (xb0[pl.ds(p, 16)] * RC
                 + xb1[pl.ds(p, 16)] * 24
                 + xb2[pl.ds(p, 16)] * 4
                 + xb3[pl.ds(p, 16)] * 2
                 + xb4[pl.ds(p, 16)])
            idx1[pl.ds(jj * 16, 16)] = v
        pltpu.async_copy(th.at[idx1], rows, sem).wait()
        pltpu.sync_copy(rows, outh.at[pl.ds(base + j * GROUP, GROUP)])
        return carry

    lax.fori_loop(0, GROUPS_PER_W, g_body, 0)


@functools.cache
def _make_sc_lookup():
    mesh = plsc.VectorSubcoreMesh(
        core_axis_name="c", subcore_axis_name="s",
        num_cores=NC, num_subcores=NS)
    return pl.kernel(
        _sc_lookup_body,
        out_type=jax.ShapeDtypeStruct((N_PAD, EMB), jnp.float32),
        mesh=mesh,
        scratch_types=[
            pltpu.VMEM((PER_W,), jnp.int32),
            pltpu.VMEM((PER_W,), jnp.int32),
            pltpu.VMEM((PER_W,), jnp.int32),
            pltpu.VMEM((PER_W,), jnp.int32),
            pltpu.VMEM((PER_W,), jnp.int32),
            pltpu.VMEM((GROUP,), jnp.int32),
            pltpu.VMEM((GROUP, EMB), jnp.float32),
            pltpu.SemaphoreType.DMA,
        ],
    )


def kernel(x, W0, W1, W2, W3, W4):
    n = x.shape[0]
    x = x.astype(jnp.int32)
    table = _build_table(W0, W1, W2, W3, W4)
    xt = jnp.pad(x.T, ((0, 0), (0, N_PAD - n)))
    out = _make_sc_lookup()(xt[0], xt[1], xt[2], xt[3], xt[4], table)
    return out[:n]


# speedup vs baseline: 3.5481x; 1.0298x over previous
import functools

import jax
import jax.numpy as jnp
from jax import lax
from jax.experimental import pallas as pl
from jax.experimental.pallas import tpu as pltpu
from jax.experimental.pallas import tpu_sc as plsc

EMB = 128
R0 = 119
RC = 288
RF = R0 * RC
MAX_NORM = 10.0

NC = 2
NS = 16
NW = NC * NS
GROUP = 128
GROUPS_PER_W = 25
PER_W = GROUP * GROUPS_PER_W
N_PAD = NW * PER_W

W0_BLK = 8


def _renorm(w):
    norm = jnp.sqrt(jnp.sum(w * w, axis=-1, keepdims=True))
    scale = jnp.where(norm > MAX_NORM, MAX_NORM / (norm + 1e-7), 1.0)
    return w * scale


def _build_body(w0_ref, w1_ref, w2_ref, w3_ref, w4_ref, out_ref, combo_ref):
    i = pl.program_id(0)

    @pl.when(i == 0)
    def _():
        r1 = _renorm(w1_ref[...])
        r2 = _renorm(w2_ref[...])
        r3 = _renorm(w3_ref[...])
        r4 = _renorm(w4_ref[...])
        r34 = jnp.concatenate([r3[0:1] + r4, r3[1:2] + r4], axis=0)
        r234 = jnp.concatenate([r2[k:k + 1] + r34 for k in range(6)], axis=0)
        combo_ref[...] = jnp.concatenate(
            [r1[a:a + 1] + r234 for a in range(12)], axis=0)

    r0 = _renorm(w0_ref[...])
    out_ref[...] = r0[:, None, :] + combo_ref[...][None, :, :]


def _build_table(W0, W1, W2, W3, W4):
    grid = (pl.cdiv(R0, W0_BLK),)
    out = pl.pallas_call(
        _build_body,
        grid=grid,
        in_specs=[
            pl.BlockSpec((W0_BLK, EMB), lambda i: (i, 0)),
            pl.BlockSpec((12, EMB), lambda i: (0, 0)),
            pl.BlockSpec((6, EMB), lambda i: (0, 0)),
            pl.BlockSpec((2, EMB), lambda i: (0, 0)),
            pl.BlockSpec((2, EMB), lambda i: (0, 0)),
        ],
        out_specs=pl.BlockSpec((W0_BLK, RC, EMB), lambda i: (i, 0, 0)),
        out_shape=jax.ShapeDtypeStruct((R0, RC, EMB), jnp.float32),
        scratch_shapes=[pltpu.VMEM((RC, EMB), jnp.float32)],
    )(W0, W1, W2, W3, W4)
    return out.reshape(RF, EMB)


DEPTH = 3


def _sc_lookup_body(x0h, x1h, x2h, x3h, x4h, th, outh,
                    xb0, xb1, xb2, xb3, xb4, idxb, rows, gsem, osem):
    wid = lax.axis_index("s") * NC + lax.axis_index("c")
    base = wid * PER_W
    xcps = [pltpu.async_copy(xh.at[pl.ds(base, PER_W)], xb, gsem)
            for xh, xb in ((x0h, xb0), (x1h, xb1), (x2h, xb2),
                           (x3h, xb3), (x4h, xb4))]
    for cp in xcps:
        cp.wait()

    def compute_idx(j, p):
        for jj in range(GROUP // 16):
            q = j * GROUP + jj * 16
            v = (xb0[pl.ds(q, 16)] * RC
                 + xb1[pl.ds(q, 16)] * 24
                 + xb2[pl.ds(q, 16)] * 4
                 + xb3[pl.ds(q, 16)] * 2
                 + xb4[pl.ds(q, 16)])
            idxb[p, pl.ds(jj * 16, 16)] = v

    def wait_one_gather():
        pltpu.make_async_copy(
            outh.at[pl.ds(base, GROUP)], rows.at[0], gsem).wait()

    def wait_one_out():
        pltpu.make_async_copy(
            rows.at[0], outh.at[pl.ds(base, GROUP)], osem).wait()

    def g_body(j, carry):
        p = lax.rem(j, DEPTH)

        @pl.when(j >= DEPTH)
        def _():
            wait_one_out()

        compute_idx(j, p)
        pltpu.async_copy(th.at[idxb.at[p]], rows.at[p], gsem)

        @pl.when(j >= 1)
        def _():
            pm1 = lax.rem(j - 1, DEPTH)
            wait_one_gather()
            pltpu.async_copy(
                rows.at[pm1], outh.at[pl.ds(base + (j - 1) * GROUP, GROUP)],
                osem)

        return carry

    lax.fori_loop(0, GROUPS_PER_W, g_body, 0)
    last = GROUPS_PER_W - 1
    wait_one_gather()
    pltpu.async_copy(
        rows.at[last % DEPTH],
        outh.at[pl.ds(base + last * GROUP, GROUP)], osem)
    for _ in range(DEPTH):
        wait_one_out()


@functools.cache
def _make_sc_lookup():
    mesh = plsc.VectorSubcoreMesh(
        core_axis_name="c", subcore_axis_name="s",
        num_cores=NC, num_subcores=NS)
    return pl.kernel(
        _sc_lookup_body,
        out_type=jax.ShapeDtypeStruct((N_PAD, EMB), jnp.float32),
        mesh=mesh,
        scratch_types=[
            pltpu.VMEM((PER_W,), jnp.int32),
            pltpu.VMEM((PER_W,), jnp.int32),
            pltpu.VMEM((PER_W,), jnp.int32),
            pltpu.VMEM((PER_W,), jnp.int32),
            pltpu.VMEM((PER_W,), jnp.int32),
            pltpu.VMEM((DEPTH, GROUP), jnp.int32),
            pltpu.VMEM((DEPTH, GROUP, EMB), jnp.float32),
            pltpu.SemaphoreType.DMA,
            pltpu.SemaphoreType.DMA,
        ],
    )


def kernel(x, W0, W1, W2, W3, W4):
    n = x.shape[0]
    x = x.astype(jnp.int32)
    table = _build_table(W0, W1, W2, W3, W4)
    xt = jnp.pad(x.T, ((0, 0), (0, N_PAD - n)))
    out = _make_sc_lookup()(xt[0], xt[1], xt[2], xt[3], xt[4], table)
    return out[:n]


# speedup vs baseline: 3.5500x; 1.0005x over previous
import functools

import jax
import jax.numpy as jnp
from jax import lax
from jax.experimental import pallas as pl
from jax.experimental.pallas import tpu as pltpu
from jax.experimental.pallas import tpu_sc as plsc

EMB = 128
R0 = 119
RC = 288
RF = R0 * RC
MAX_NORM = 10.0

NC = 2
NS = 16
NW = NC * NS
GROUP = 128
GROUPS_PER_W = 25
PER_W = GROUP * GROUPS_PER_W
N_PAD = NW * PER_W

W0_BLK = 8


def _renorm(w):
    norm = jnp.sqrt(jnp.sum(w * w, axis=-1, keepdims=True))
    scale = jnp.where(norm > MAX_NORM, MAX_NORM / (norm + 1e-7), 1.0)
    return w * scale


def _build_body(w0_ref, w1_ref, w2_ref, w3_ref, w4_ref, out_ref, combo_ref):
    i = pl.program_id(0)

    @pl.when(i == 0)
    def _():
        r1 = _renorm(w1_ref[...])
        r2 = _renorm(w2_ref[...])
        r3 = _renorm(w3_ref[...])
        r4 = _renorm(w4_ref[...])
        r34 = jnp.concatenate([r3[0:1] + r4, r3[1:2] + r4], axis=0)
        r234 = jnp.concatenate([r2[k:k + 1] + r34 for k in range(6)], axis=0)
        combo_ref[...] = jnp.concatenate(
            [r1[a:a + 1] + r234 for a in range(12)], axis=0)

    r0 = _renorm(w0_ref[...])
    out_ref[...] = r0[:, None, :] + combo_ref[...][None, :, :]


def _build_table(W0, W1, W2, W3, W4):
    grid = (pl.cdiv(R0, W0_BLK),)
    out = pl.pallas_call(
        _build_body,
        grid=grid,
        in_specs=[
            pl.BlockSpec((W0_BLK, EMB), lambda i: (i, 0)),
            pl.BlockSpec((12, EMB), lambda i: (0, 0)),
            pl.BlockSpec((6, EMB), lambda i: (0, 0)),
            pl.BlockSpec((2, EMB), lambda i: (0, 0)),
            pl.BlockSpec((2, EMB), lambda i: (0, 0)),
        ],
        out_specs=pl.BlockSpec((W0_BLK, RC, EMB), lambda i: (i, 0, 0)),
        out_shape=jax.ShapeDtypeStruct((R0, RC, EMB), jnp.float32),
        scratch_shapes=[pltpu.VMEM((RC, EMB), jnp.float32)],
    )(W0, W1, W2, W3, W4)
    return out.reshape(RF, EMB)


DEPTH = 5
GLAG = 2


def _sc_lookup_body(x0h, x1h, x2h, x3h, x4h, th, outh,
                    xb0, xb1, xb2, xb3, xb4, idxb, rows, gsem, osem):
    wid = lax.axis_index("s") * NC + lax.axis_index("c")
    base = wid * PER_W
    xcps = [pltpu.async_copy(xh.at[pl.ds(base, PER_W)], xb, gsem)
            for xh, xb in ((x0h, xb0), (x1h, xb1), (x2h, xb2),
                           (x3h, xb3), (x4h, xb4))]
    for cp in xcps:
        cp.wait()

    def compute_idx(j, p):
        for jj in range(GROUP // 16):
            q = j * GROUP + jj * 16
            v = (xb0[pl.ds(q, 16)] * RC
                 + xb1[pl.ds(q, 16)] * 24
                 + xb2[pl.ds(q, 16)] * 4
                 + xb3[pl.ds(q, 16)] * 2
                 + xb4[pl.ds(q, 16)])
            idxb[p, pl.ds(jj * 16, 16)] = v

    def wait_one_gather():
        pltpu.make_async_copy(
            outh.at[pl.ds(base, GROUP)], rows.at[0], gsem).wait()

    def wait_one_out():
        pltpu.make_async_copy(
            rows.at[0], outh.at[pl.ds(base, GROUP)], osem).wait()

    def g_body(j, carry):
        p = lax.rem(j, DEPTH)

        @pl.when(j >= DEPTH)
        def _():
            wait_one_out()

        compute_idx(j, p)
        pltpu.async_copy(th.at[idxb.at[p]], rows.at[p], gsem)

        @pl.when(j >= GLAG)
        def _():
            pmg = lax.rem(j - GLAG, DEPTH)
            wait_one_gather()
            pltpu.async_copy(
                rows.at[pmg], outh.at[pl.ds(base + (j - GLAG) * GROUP, GROUP)],
                osem)

        return carry

    lax.fori_loop(0, GROUPS_PER_W, g_body, 0)
    for k in range(GLAG):
        jj = GROUPS_PER_W - GLAG + k
        wait_one_gather()
        pltpu.async_copy(
            rows.at[jj % DEPTH],
            outh.at[pl.ds(base + jj * GROUP, GROUP)], osem)
    for _ in range(DEPTH):
        wait_one_out()


@functools.cache
def _make_sc_lookup():
    mesh = plsc.VectorSubcoreMesh(
        core_axis_name="c", subcore_axis_name="s",
        num_cores=NC, num_subcores=NS)
    return pl.kernel(
        _sc_lookup_body,
        out_type=jax.ShapeDtypeStruct((N_PAD, EMB), jnp.float32),
        mesh=mesh,
        scratch_types=[
            pltpu.VMEM((PER_W,), jnp.int32),
            pltpu.VMEM((PER_W,), jnp.int32),
            pltpu.VMEM((PER_W,), jnp.int32),
            pltpu.VMEM((PER_W,), jnp.int32),
            pltpu.VMEM((PER_W,), jnp.int32),
            pltpu.VMEM((DEPTH, GROUP), jnp.int32),
            pltpu.VMEM((DEPTH, GROUP, EMB), jnp.float32),
            pltpu.SemaphoreType.DMA,
            pltpu.SemaphoreType.DMA,
        ],
    )


def kernel(x, W0, W1, W2, W3, W4):
    n = x.shape[0]
    x = x.astype(jnp.int32)
    table = _build_table(W0, W1, W2, W3, W4)
    xt = jnp.pad(x.T, ((0, 0), (0, N_PAD - n)))
    out = _make_sc_lookup()(xt[0], xt[1], xt[2], xt[3], xt[4], table)
    return out[:n]


# speedup vs baseline: 6.7248x; 1.8943x over previous
import functools

import jax
import jax.numpy as jnp
from jax import lax
from jax.experimental import pallas as pl
from jax.experimental.pallas import tpu as pltpu
from jax.experimental.pallas import tpu_sc as plsc

EMB = 128
R0 = 119
RC = 288
MAX_NORM = 10.0

NC = 2
NS = 16
NW = NC * NS
GROUP = 128
GROUPS_PER_W = 25
PER_W = GROUP * GROUPS_PER_W
N_PAD = NW * PER_W

DEPTH = 2


def _renorm(w):
    norm = jnp.sqrt(jnp.sum(w * w, axis=-1, keepdims=True))
    scale = jnp.where(norm > MAX_NORM, MAX_NORM / (norm + 1e-7), 1.0)
    return w * scale


def _build_body(w0_ref, w1_ref, w2_ref, w3_ref, w4_ref, t0_ref, t1_ref):
    t0_ref[...] = _renorm(w0_ref[...])
    r1 = _renorm(w1_ref[...])
    r2 = _renorm(w2_ref[...])
    r3 = _renorm(w3_ref[...])
    r4 = _renorm(w4_ref[...])
    r34 = jnp.concatenate([r3[0:1] + r4, r3[1:2] + r4], axis=0)
    r234 = jnp.concatenate([r2[k:k + 1] + r34 for k in range(6)], axis=0)
    t1_ref[...] = jnp.concatenate(
        [r1[a:a + 1] + r234 for a in range(12)], axis=0)


def _build_tables(W0, W1, W2, W3, W4):
    return pl.pallas_call(
        _build_body,
        out_shape=[
            jax.ShapeDtypeStruct((R0, EMB), jnp.float32),
            jax.ShapeDtypeStruct((RC, EMB), jnp.float32),
        ],
    )(W0, W1, W2, W3, W4)


def _sc_lookup_body(x0h, x1h, x2h, x3h, x4h, t0h, t1h, outh,
                    xb0, xb1, xb2, xb3, xb4, t0b, t1b, rows3, osem):
    wid = lax.axis_index("s") * NC + lax.axis_index("c")
    base = wid * PER_W
    cps = [pltpu.async_copy(xh.at[pl.ds(base, PER_W)], xb, osem)
           for xh, xb in ((x0h, xb0), (x1h, xb1), (x2h, xb2),
                          (x3h, xb3), (x4h, xb4))]
    cps.append(pltpu.async_copy(t0h, t0b, osem))
    cps.append(pltpu.async_copy(t1h, t1b, osem))
    for cp in cps:
        cp.wait()

    iota16 = lax.iota(jnp.int32, 16)
    ones = jnp.full((16,), 1, jnp.int32)
    mask127 = jnp.full((16,), EMB - 1, jnp.int32)

    def wait_one_out():
        pltpu.make_async_copy(
            rows3.at[0], outh.at[pl.ds(base, GROUP)], osem).wait()

    def j_body(j, carry):
        p = lax.rem(j, DEPTH)

        @pl.when(j >= DEPTH)
        def _():
            wait_one_out()

        pv = jnp.full((16,), p, jnp.int32)

        def g_body(g, carry2):
            q = j * GROUP + g * 16
            row_a = xb0[pl.ds(q, 16)]
            row_b = (xb1[pl.ds(q, 16)] * 24
                     + xb2[pl.ds(q, 16)] * 4
                     + xb3[pl.ds(q, 16)] * 2
                     + xb4[pl.ds(q, 16)])
            nodev = g * 16 + iota16
            cv = nodev & mask127
            for _c in range(EMB):
                va = plsc.load_gather(t0b, [row_a, cv])
                vb = plsc.load_gather(t1b, [row_b, cv])
                plsc.store_scatter(rows3, [pv, nodev, cv], va + vb)
                cv = (cv + ones) & mask127
            return carry2

        lax.fori_loop(0, GROUP // 16, g_body, 0)
        pltpu.async_copy(
            rows3.at[p], outh.at[pl.ds(base + j * GROUP, GROUP)], osem)
        return carry

    lax.fori_loop(0, GROUPS_PER_W, j_body, 0)
    for _ in range(DEPTH):
        wait_one_out()


@functools.cache
def _make_sc_lookup():
    mesh = plsc.VectorSubcoreMesh(
        core_axis_name="c", subcore_axis_name="s",
        num_cores=NC, num_subcores=NS)
    return pl.kernel(
        _sc_lookup_body,
        out_type=jax.ShapeDtypeStruct((N_PAD, EMB), jnp.float32),
        mesh=mesh,
        compiler_params=pltpu.CompilerParams(needs_layout_passes=False),
        scratch_types=[
            pltpu.VMEM((PER_W,), jnp.int32),
            pltpu.VMEM((PER_W,), jnp.int32),
            pltpu.VMEM((PER_W,), jnp.int32),
            pltpu.VMEM((PER_W,), jnp.int32),
            pltpu.VMEM((PER_W,), jnp.int32),
            pltpu.VMEM((R0, EMB), jnp.float32),
            pltpu.VMEM((RC, EMB), jnp.float32),
            pltpu.VMEM((DEPTH, GROUP, EMB), jnp.float32),
            pltpu.SemaphoreType.DMA,
        ],
    )


def kernel(x, W0, W1, W2, W3, W4):
    n = x.shape[0]
    x = x.astype(jnp.int32)
    t0, t1 = _build_tables(W0, W1, W2, W3, W4)
    xt = jnp.pad(x.T, ((0, 0), (0, N_PAD - n)))
    out = _make_sc_lookup()(xt[0], xt[1], xt[2], xt[3], xt[4], t0, t1)
    return out[:n]
